# baseline (speedup 1.0000x reference)
import jax
import jax.numpy as jnp
from jax import lax
from jax.experimental import pallas as pl
from jax.experimental.pallas import tpu as pltpu
from jax.experimental.pallas import tpu_sc as plsc

N = 10000
E = 320000
D = 128

NC = 2
NS = 16
NW = NC * NS
NPAD = 10240
SL1 = NPAD // NS
RPT = NPAD // NS
RST = 128
CH = 128
SUP = 8
NSUP = 10
NCH = SUP * NSUP
SUPE = SUP * CH
EPT = NCH * CH
PADE = EPT * NW - E
NPC = NPAD // D

_mesh = plsc.VectorSubcoreMesh(core_axis_name="c", subcore_axis_name="s")


def _deg_body(cols3, deg_out, idx2_v, ones_v, zero_v, stage_v, deg_sh):
    c = lax.axis_index("c")
    s = lax.axis_index("s")
    w = c * NS + s
    one16 = jnp.ones((16,), jnp.float32)
    zer16 = jnp.zeros((16,), jnp.float32)
    for i in range(CH // 16):
        ones_v[pl.ds(i * 16, 16)] = one16
    for i in range(SL1 // 16):
        zero_v[pl.ds(i * 16, 16)] = zer16
    pltpu.sync_copy(cols3.at[w], idx2_v)
    obase = pl.multiple_of(s * SL1, 8)
    pltpu.sync_copy(zero_v, deg_sh.at[pl.ds(obase, SL1)])
    plsc.subcore_barrier()

    def step(k, carry):
        pltpu.sync_copy(ones_v, deg_sh.at[idx2_v.at[k]], add=True)
        return carry

    lax.fori_loop(0, NCH, step, 0)
    plsc.subcore_barrier()

    pltpu.sync_copy(deg_sh.at[pl.ds(obase, SL1)], stage_v)
    ob = pl.multiple_of(c * NPAD + obase, 8)
    pltpu.sync_copy(stage_v, deg_out.at[pl.ds(ob, SL1)])


def _degrees(cols3):
    return pl.kernel(
        _deg_body,
        out_type=jax.ShapeDtypeStruct((NC * NPAD,), jnp.float32),
        mesh=_mesh,
        scratch_types=[
            pltpu.VMEM((NCH, CH), jnp.int32),
            pltpu.VMEM((CH,), jnp.float32),
            pltpu.VMEM((SL1,), jnp.float32),
            pltpu.VMEM((SL1,), jnp.float32),
            pltpu.VMEM_SHARED((NPAD,), jnp.float32),
        ],
    )(cols3)


def _msg_body(scaled1, rows1, cols1, zrows, acc_out, ic0, ic1, ir0, ir1,
              b0, b1, gs0, gs1, isem, acc_sh):
    c = lax.axis_index("c")
    s = lax.axis_index("s")
    w = c * NS + s
    bufs = (b0, b1)
    gsems = (gs0, gs1)
    ics = (ic0, ic1)
    irs = (ir0, ir1)
    wbase = pl.multiple_of(w * EPT, 8)

    pltpu.sync_copy(zrows, b0)
    rbase = s * RPT
    for j in range(RPT // RST):
        pltpu.sync_copy(b0, acc_sh.at[pl.ds(rbase + j * RST, RST)])
    plsc.subcore_barrier()

    pltpu.sync_copy(cols1.at[pl.ds(wbase, SUPE)], ic0)
    pltpu.sync_copy(rows1.at[pl.ds(wbase, SUPE)], ir0)
    nb = pl.multiple_of(wbase + SUPE, 8)
    pltpu.async_copy(cols1.at[pl.ds(nb, SUPE)], ic1, isem)
    pltpu.async_copy(rows1.at[pl.ds(nb, SUPE)], ir1, isem)
    pltpu.async_copy(scaled1.at[ic0.at[pl.ds(0, CH)]], b0, gs0)
    pltpu.async_copy(scaled1.at[ic0.at[pl.ds(CH, CH)]], b1, gs1)

    def super_block(v, sp):
        u = 2 * v + sp
        ic, ir = ics[sp], irs[sp]
        icn = ics[1 - sp]

        def chunk_pair(i, carry):
            for b in range(2):
                j = 2 * i + b
                pltpu.make_async_copy(scaled1.at[ic.at[pl.ds(0, CH)]],
                                      bufs[b], gsems[b]).wait()
                pltpu.sync_copy(bufs[b],
                                acc_sh.at[ir.at[pl.ds(j * CH, CH)]],
                                add=True)
                pltpu.async_copy(scaled1.at[ic.at[pl.ds((j + 2) * CH, CH)]],
                                 bufs[b], gsems[b])
            return carry

        lax.fori_loop(0, SUP // 2 - 1, chunk_pair, 0)

        pltpu.make_async_copy(scaled1.at[ic.at[pl.ds(0, CH)]], b0,
                              gs0).wait()
        pltpu.sync_copy(b0, acc_sh.at[ir.at[pl.ds((SUP - 2) * CH, CH)]],
                        add=True)

        @pl.when(u + 1 < NSUP)
        def _():
            pltpu.make_async_copy(cols1.at[pl.ds(wbase, SUPE)], icn,
                                  isem).wait()
            pltpu.make_async_copy(cols1.at[pl.ds(wbase, SUPE)], icn,
                                  isem).wait()
            pltpu.async_copy(scaled1.at[icn.at[pl.ds(0, CH)]], b0, gs0)

        pltpu.make_async_copy(scaled1.at[ic.at[pl.ds(0, CH)]], b1,
                              gs1).wait()
        pltpu.sync_copy(b1, acc_sh.at[ir.at[pl.ds((SUP - 1) * CH, CH)]],
                        add=True)

        @pl.when(u + 1 < NSUP)
        def _():
            pltpu.async_copy(scaled1.at[icn.at[pl.ds(CH, CH)]], b1, gs1)

        @pl.when(u + 2 < NSUP)
        def _():
            fb = pl.multiple_of(wbase + (u + 2) * SUPE, 8)
            pltpu.async_copy(cols1.at[pl.ds(fb, SUPE)], ic, isem)
            pltpu.async_copy(rows1.at[pl.ds(fb, SUPE)], ir, isem)

    def super_pair(v, carry):
        super_block(v, 0)
        super_block(v, 1)
        return carry

    lax.fori_loop(0, NSUP // 2, super_pair, 0)
    plsc.subcore_barrier()

    for j in range(RPT // RST):
        r0 = rbase + j * RST
        pltpu.sync_copy(acc_sh.at[pl.ds(r0, RST)], b0)
        pltpu.sync_copy(b0, acc_out.at[c, pl.ds(r0, RST)])


def _message_pass(scaled1, rows1, cols1, zrows):
    return pl.kernel(
        _msg_body,
        out_type=jax.ShapeDtypeStruct((NC, NPAD, D), jnp.float32),
        mesh=_mesh,
        scratch_types=[
            pltpu.VMEM((SUPE,), jnp.int32),
            pltpu.VMEM((SUPE,), jnp.int32),
            pltpu.VMEM((SUPE,), jnp.int32),
            pltpu.VMEM((SUPE,), jnp.int32),
            pltpu.VMEM((CH, D), jnp.float32),
            pltpu.VMEM((CH, D), jnp.float32),
            pltpu.SemaphoreType.DMA,
            pltpu.SemaphoreType.DMA,
            pltpu.SemaphoreType.DMA,
            pltpu.VMEM_SHARED((NPAD, D), jnp.float32),
        ],
    )(scaled1, rows1, cols1, zrows)


def _scale_body(deg_ref, emb_ref, dis_ref, scaled_ref):
    deg = jnp.maximum(deg_ref[0] + deg_ref[1], 1.0)
    dis = lax.rsqrt(deg)
    dis_ref[...] = dis
    disn = dis.reshape(NPAD)[:N].reshape(N, 1)
    scaled_ref[...] = disn * emb_ref[...]


def _scale(deg3, emb):
    return pl.pallas_call(
        _scale_body,
        out_shape=[
            jax.ShapeDtypeStruct((NPC, D), jnp.float32),
            jax.ShapeDtypeStruct((N, D), jnp.float32),
        ],
    )(deg3, emb)


def _combine_body(eps_ref, dis_ref, acc_ref, scaled_ref, out_ref):
    epsp1 = 1.0 + eps_ref[0]
    disn = dis_ref[...].reshape(NPAD)[:N].reshape(N, 1)
    acc = acc_ref[0, :N, :] + acc_ref[1, :N, :]
    out_ref[...] = disn * (acc + epsp1 * scaled_ref[...])


def _combine(eps, dis, acc, scaled):
    return pl.pallas_call(
        _combine_body,
        in_specs=[
            pl.BlockSpec(memory_space=pltpu.SMEM),
            pl.BlockSpec(),
            pl.BlockSpec(),
            pl.BlockSpec(),
        ],
        out_shape=jax.ShapeDtypeStruct((N, D), jnp.float32),
    )(eps, dis, acc, scaled)


def kernel(pos_emb, neg_emb, pos_edge_index, neg_edge_index, eps):
    pos_ei = pos_edge_index.astype(jnp.int32)
    neg_ei = neg_edge_index.astype(jnp.int32)

    padg = jnp.arange(PADE, dtype=jnp.int32) % 240
    padh = N + (jnp.arange(PADE, dtype=jnp.int32) % (NPAD - N))

    def one_sign(emb, ei):
        rows1 = jnp.concatenate([ei[0], padh])
        colsd = jnp.concatenate([ei[1], padh]).reshape(NW, NCH, CH)
        colsg = jnp.concatenate([ei[1], padg])
        degp = _degrees(colsd)
        dis, scaled = _scale(degp.reshape(NC, NPC, D), emb)
        zrows = jnp.zeros((RST, D), jnp.float32)
        acc = _message_pass(scaled, rows1, colsg, zrows)
        return _combine(eps, dis, acc, scaled)

    out_pos = one_sign(pos_emb, pos_ei)
    out_neg = one_sign(neg_emb, neg_ei)
    return (out_pos, out_neg)


# speedup vs baseline: 1.0590x; 1.0590x over previous
import jax
import jax.numpy as jnp
from jax import lax
from jax.experimental import pallas as pl
from jax.experimental.pallas import tpu as pltpu
from jax.experimental.pallas import tpu_sc as plsc

N = 10000
E = 320000
D = 128

NC = 2
NS = 16
NPAD = 10240
SL1 = NPAD // NS
RPT = NPAD // NS
RST = 128
CH = 128
SUP = 16
NSUP = 10
NCH = SUP * NSUP
SUPE = SUP * CH
EPTP = NCH * CH
PADE = EPTP * NS - E

_mesh = plsc.VectorSubcoreMesh(core_axis_name="c", subcore_axis_name="s")


def _deg_body(cols3, deg_out, idx2_v, ones_v, zero_v, stage_v, deg_sh):
    c = lax.axis_index("c")
    s = lax.axis_index("s")
    w = c * NS + s
    one16 = jnp.ones((16,), jnp.float32)
    zer16 = jnp.zeros((16,), jnp.float32)
    for i in range(CH // 16):
        ones_v[pl.ds(i * 16, 16)] = one16
    for i in range(SL1 // 16):
        zero_v[pl.ds(i * 16, 16)] = zer16
    pltpu.sync_copy(cols3.at[w], idx2_v)
    obase = pl.multiple_of(s * SL1, 8)
    pltpu.sync_copy(zero_v, deg_sh.at[pl.ds(obase, SL1)])
    plsc.subcore_barrier()

    def step(k, carry):
        pltpu.sync_copy(ones_v, deg_sh.at[idx2_v.at[k]], add=True)
        return carry

    lax.fori_loop(0, NCH, step, 0)
    plsc.subcore_barrier()

    pltpu.sync_copy(deg_sh.at[pl.ds(obase, SL1)], stage_v)
    ob = pl.multiple_of(c * NPAD + obase, 8)
    pltpu.sync_copy(stage_v, deg_out.at[pl.ds(ob, SL1)])


def _degrees(cols3):
    return pl.kernel(
        _deg_body,
        out_type=jax.ShapeDtypeStruct((NC * NPAD,), jnp.float32),
        mesh=_mesh,
        scratch_types=[
            pltpu.VMEM((NCH, CH), jnp.int32),
            pltpu.VMEM((CH,), jnp.float32),
            pltpu.VMEM((SL1,), jnp.float32),
            pltpu.VMEM((SL1,), jnp.float32),
            pltpu.VMEM_SHARED((NPAD,), jnp.float32),
        ],
    )(cols3)


def _msg_body(scaled2, rows1, cols1, zrows, acc_out, ic0, ic1, ir0, ir1,
              b0, b1, gs0, gs1, isem, acc_sh):
    c = lax.axis_index("c")
    s = lax.axis_index("s")
    w = c * NS + s
    bufs = (b0, b1)
    gsems = (gs0, gs1)
    ics = (ic0, ic1)
    irs = (ir0, ir1)
    wbase = pl.multiple_of(w * EPTP, 8)

    pltpu.sync_copy(zrows, b0)
    rbase = s * RPT
    for j in range(RPT // RST):
        pltpu.sync_copy(b0, acc_sh.at[pl.ds(rbase + j * RST, RST)])
    plsc.subcore_barrier()

    pltpu.sync_copy(cols1.at[pl.ds(wbase, SUPE)], ic0)
    pltpu.sync_copy(rows1.at[pl.ds(wbase, SUPE)], ir0)
    nb = pl.multiple_of(wbase + SUPE, 8)
    pltpu.async_copy(cols1.at[pl.ds(nb, SUPE)], ic1, isem)
    pltpu.async_copy(rows1.at[pl.ds(nb, SUPE)], ir1, isem)
    pltpu.async_copy(scaled2.at[ic0.at[pl.ds(0, CH)]], b0, gs0)
    pltpu.async_copy(scaled2.at[ic0.at[pl.ds(CH, CH)]], b1, gs1)

    def super_block(v, sp):
        u = 2 * v + sp
        ic, ir = ics[sp], irs[sp]
        icn = ics[1 - sp]

        def chunk_pair(i, carry):
            for b in range(2):
                j = 2 * i + b
                pltpu.make_async_copy(zrows, bufs[b], gsems[b]).wait()
                pltpu.sync_copy(bufs[b],
                                acc_sh.at[ir.at[pl.ds(j * CH, CH)]],
                                add=True)
                pltpu.async_copy(scaled2.at[ic.at[pl.ds((j + 2) * CH, CH)]],
                                 bufs[b], gsems[b])
            return carry

        lax.fori_loop(0, SUP // 2 - 1, chunk_pair, 0)

        pltpu.make_async_copy(zrows, b0, gs0).wait()
        pltpu.sync_copy(b0, acc_sh.at[ir.at[pl.ds((SUP - 2) * CH, CH)]],
                        add=True)

        @pl.when(u + 1 < NSUP)
        def _():
            pltpu.make_async_copy(cols1.at[pl.ds(wbase, SUPE)], icn,
                                  isem).wait()
            pltpu.make_async_copy(cols1.at[pl.ds(wbase, SUPE)], icn,
                                  isem).wait()
            pltpu.async_copy(scaled2.at[icn.at[pl.ds(0, CH)]], b0, gs0)

        pltpu.make_async_copy(zrows, b1, gs1).wait()
        pltpu.sync_copy(b1, acc_sh.at[ir.at[pl.ds((SUP - 1) * CH, CH)]],
                        add=True)

        @pl.when(u + 1 < NSUP)
        def _():
            pltpu.async_copy(scaled2.at[icn.at[pl.ds(CH, CH)]], b1, gs1)

        @pl.when(u + 2 < NSUP)
        def _():
            fb = pl.multiple_of(wbase + (u + 2) * SUPE, 8)
            pltpu.async_copy(cols1.at[pl.ds(fb, SUPE)], ic, isem)
            pltpu.async_copy(rows1.at[pl.ds(fb, SUPE)], ir, isem)

    def super_pair(v, carry):
        super_block(v, 0)
        super_block(v, 1)
        return carry

    lax.fori_loop(0, NSUP // 2, super_pair, 0)
    plsc.subcore_barrier()

    for j in range(RPT // RST):
        r0 = rbase + j * RST
        pltpu.sync_copy(acc_sh.at[pl.ds(r0, RST)], b0)
        pltpu.sync_copy(b0, acc_out.at[c, pl.ds(r0, RST)])


def _message_pass(scaled2, rows1, cols1, zrows):
    return pl.kernel(
        _msg_body,
        out_type=jax.ShapeDtypeStruct((NC, NPAD, D), jnp.float32),
        mesh=_mesh,
        scratch_types=[
            pltpu.VMEM((SUPE,), jnp.int32),
            pltpu.VMEM((SUPE,), jnp.int32),
            pltpu.VMEM((SUPE,), jnp.int32),
            pltpu.VMEM((SUPE,), jnp.int32),
            pltpu.VMEM((CH, D), jnp.float32),
            pltpu.VMEM((CH, D), jnp.float32),
            pltpu.SemaphoreType.DMA,
            pltpu.SemaphoreType.DMA,
            pltpu.SemaphoreType.DMA,
            pltpu.VMEM_SHARED((NPAD, D), jnp.float32),
        ],
    )(scaled2, rows1, cols1, zrows)


NPC = NPAD // D


def _scale_body(deg_ref, pemb_ref, nemb_ref, dis_ref, scaled_ref):
    for c in range(NC):
        deg = jnp.maximum(deg_ref[c], 1.0)
        dis = lax.rsqrt(deg)
        dis_ref[c] = dis
        disn = dis.reshape(NPAD)[:N].reshape(N, 1)
        emb = pemb_ref[...] if c == 0 else nemb_ref[...]
        scaled_ref[c] = disn * emb


def _scale(deg3, pos_emb, neg_emb):
    return pl.pallas_call(
        _scale_body,
        out_shape=[
            jax.ShapeDtypeStruct((NC, NPC, D), jnp.float32),
            jax.ShapeDtypeStruct((NC, N, D), jnp.float32),
        ],
    )(deg3, pos_emb, neg_emb)


def _combine_body(eps_ref, dis_ref, acc_ref, scaled_ref, out_ref):
    epsp1 = 1.0 + eps_ref[0]
    disn = dis_ref[0].reshape(NPAD)[:N].reshape(N, 1)
    out_ref[0] = disn * (acc_ref[0] + epsp1 * scaled_ref[0])


def _combine(eps, dis3, acc2, scaled2):
    return pl.pallas_call(
        _combine_body,
        grid=(NC,),
        in_specs=[
            pl.BlockSpec(memory_space=pltpu.SMEM),
            pl.BlockSpec((1, NPC, D), lambda c: (c, 0, 0)),
            pl.BlockSpec((1, N, D), lambda c: (c, 0, 0)),
            pl.BlockSpec((1, N, D), lambda c: (c, 0, 0)),
        ],
        out_specs=pl.BlockSpec((1, N, D), lambda c: (c, 0, 0)),
        out_shape=jax.ShapeDtypeStruct((NC, N, D), jnp.float32),
    )(eps, dis3, acc2, scaled2)


def kernel(pos_emb, neg_emb, pos_edge_index, neg_edge_index, eps):
    pos_ei = pos_edge_index.astype(jnp.int32)
    neg_ei = neg_edge_index.astype(jnp.int32)

    padv = (N + (jnp.arange(PADE, dtype=jnp.int32) % (NPAD - N)))
    rows_flat = jnp.concatenate([pos_ei[0], padv, neg_ei[0], padv])
    colsd_flat = jnp.concatenate(
        [pos_ei[1], padv, neg_ei[1], padv]).reshape(NC * NS, NCH, CH)
    colsg_flat = jnp.concatenate([pos_ei[1], padv, neg_ei[1] + N, padv])
    deg2 = _degrees(colsd_flat)
    dis2, scaled2 = _scale(deg2.reshape(NC, NPC, D), pos_emb, neg_emb)
    scaled_flat = scaled2.reshape(NC * N, D)
    zrows = jnp.zeros((RST, D), jnp.float32)
    acc2 = _message_pass(scaled_flat, rows_flat, colsg_flat, zrows)
    out2 = _combine(eps, dis2, acc2, scaled2)
    return (out2[0], out2[1])
